# Initial kernel scaffold; baseline (speedup 1.0000x reference)
import jax
import jax.numpy as jnp
from jax.experimental import pallas as pl


def kernel(x1, edge_index1, edge_attr1, batch1, x2, edge_index2, edge_attr2, batch2, params):
    raise NotImplementedError("write your pallas kernel here")


# speedup vs baseline: 30.3815x; 30.3815x over previous
import functools

import jax
import jax.numpy as jnp
from jax import lax
from jax.experimental import pallas as pl
from jax.experimental.pallas import tpu as pltpu
from jax.experimental.pallas import tpu_sc as plsc

N_GR = 16
RN = 200
EPER = 8000
NB = 2 * N_GR
ASZ = RN * RN
REG = ASZ + 8
CW = 128
NCHUNK = (EPER + CW - 1) // CW
EPAD = NCHUNK * CW
K1 = RN // 2
K2 = K1 // 2
LAT = 64

_HI = lax.Precision.HIGHEST


def _dot(a, b):
    return jnp.dot(a, b, precision=_HI, preferred_element_type=jnp.float32)


def _dotl(a, b):
    return jnp.dot(a.astype(jnp.bfloat16), b.astype(jnp.bfloat16),
                   preferred_element_type=jnp.float32)


def _adj_body(idx_hbm, vals_hbm, zeros_hbm, out_hbm, idx_v, vals_v, buf_v,
              acc_sh, sem):
    c = lax.axis_index("c")
    s = lax.axis_index("s")
    b = c * N_GR + s
    base = s * REG

    pltpu.sync_copy(idx_hbm.at[b], idx_v)
    pltpu.sync_copy(vals_hbm.at[b], vals_v)
    pltpu.sync_copy(zeros_hbm, buf_v)
    pltpu.sync_copy(buf_v, acc_sh.at[pl.ds(base, ASZ)])

    def fire(j, carry):
        pltpu.async_copy(vals_v.at[j], acc_sh.at[idx_v.at[j]], sem,
                         add=True).wait()
        return carry

    lax.fori_loop(0, NCHUNK, fire, 0)

    pltpu.sync_copy(acc_sh.at[pl.ds(base, ASZ)], buf_v)
    pltpu.sync_copy(buf_v, out_hbm.at[b])


def _build_adj(idx, vals, zeros):
    mesh = plsc.VectorSubcoreMesh(core_axis_name="c", subcore_axis_name="s")
    f = pl.kernel(
        _adj_body,
        out_type=jax.ShapeDtypeStruct((NB, ASZ), jnp.float32),
        mesh=mesh,
        scratch_types=[
            pltpu.VMEM((NCHUNK, CW), jnp.int32),
            pltpu.VMEM((NCHUNK, CW), jnp.float32),
            pltpu.VMEM((ASZ,), jnp.float32),
            pltpu.VMEM_SHARED((N_GR * REG,), jnp.float32),
            pltpu.SemaphoreType.DMA,
        ],
    )
    return f(idx, vals, zeros)


def _col2row(v, n):
    i = lax.broadcasted_iota(jnp.int32, (n, n), 0)
    j = lax.broadcasted_iota(jnp.int32, (n, n), 1)
    eye = (i == j).astype(jnp.float32)
    return _dot(jnp.ones((1, n), jnp.float32), v * eye)


def _cheb(A_norm, x, W_ref, b_row, lmul_dot):
    tx1 = -lmul_dot(A_norm, x)
    tx2 = -2.0 * lmul_dot(A_norm, tx1) - x
    return (_dotl(x, W_ref[0]) + _dotl(tx1, W_ref[1]) + _dotl(tx2, W_ref[2])
            + b_row)


def _norm_adj(A, n):
    deg = jnp.sum(A, axis=1, keepdims=True)
    pos = deg > 0.0
    dinv = jnp.where(pos, lax.rsqrt(jnp.where(pos, deg, 1.0)), 0.0)
    return A * dinv * _col2row(dinv, n)


def _select(score, n, k):
    i = lax.broadcasted_iota(jnp.int32, (n, n), 0)
    j = lax.broadcasted_iota(jnp.int32, (n, n), 1)
    s_row = _col2row(score, n)
    beats = (s_row > score) | ((s_row == score) & (j < i))
    rank = jnp.sum(beats.astype(jnp.float32), axis=1, keepdims=True)
    kept = rank < float(k)
    keptf = kept.astype(jnp.float32)
    lstrict = (j < i).astype(jnp.float32)
    posn = _dot(lstrict, keptf)
    pos_row = _col2row(posn, n)
    kept_row = _col2row(keptf, n)
    pk = lax.broadcasted_iota(jnp.int32, (k, 1), 0).astype(jnp.float32)
    S = ((pk == pos_row) & (kept_row > 0.5)).astype(jnp.float32)
    rk = lax.broadcasted_iota(jnp.int32, (1, k), 1).astype(jnp.float32)
    St = ((posn == rk) & kept).astype(jnp.float32)
    return S, St


def _sigmoid(z):
    return 1.0 / (1.0 + jnp.exp(-z))


def _score(x, pw_ref, n):
    pw = pw_ref[...]
    wnorm = jnp.sqrt(jnp.sum(pw * pw))
    i = lax.broadcasted_iota(jnp.int32, (LAT, LAT), 0)
    j = lax.broadcasted_iota(jnp.int32, (LAT, LAT), 1)
    eye = (i == j).astype(jnp.float32)
    pw_col = _dot(eye * pw, jnp.ones((LAT, 1), jnp.float32))
    return _sigmoid(_dotl(x, pw_col) / wnorm)


def _graph_body(a_ref, x_ref, w1_ref, b1_ref, pw1_ref, w2_ref, b2_ref, pw2_ref,
                h_ref):
    A = a_ref[0]
    x = x_ref[0]

    xc = _cheb(_norm_adj(A, RN), x, w1_ref, b1_ref[...], _dot)

    s1 = _score(xc, pw1_ref, RN)
    S1, St1 = _select(s1, RN, K1)
    xp = _dot(S1, xc * s1)
    Ap = _dot(S1, _dot(A, St1))

    h_ref[0, 0:1, 0:64] = jnp.max(xp, axis=0, keepdims=True)
    h_ref[0, 0:1, 64:128] = jnp.sum(xp, axis=0, keepdims=True) / float(K1)

    i = lax.broadcasted_iota(jnp.int32, (K1, K1), 0)
    j = lax.broadcasted_iota(jnp.int32, (K1, K1), 1)
    eye_k = (i == j).astype(jnp.float32)
    aaug = Ap + eye_k
    A2 = _dotl(aaug, aaug) * (1.0 - eye_k)

    xc2 = _cheb(_norm_adj(A2, K1), xp, w2_ref, b2_ref[...], _dotl)

    s2 = _score(xc2, pw2_ref, K1)
    S2, _ = _select(s2, K1, K2)
    xp2 = _dot(S2, xc2 * s2)

    h_ref[0, 0:1, 128:192] = jnp.max(xp2, axis=0, keepdims=True)
    h_ref[0, 0:1, 192:256] = jnp.sum(xp2, axis=0, keepdims=True) / float(K2)


def _graph_tc(A_all, X_all, w1, b1, pw1, w2, b2, pw2):
    return pl.pallas_call(
        _graph_body,
        grid=(NB,),
        in_specs=[
            pl.BlockSpec((1, RN, RN), lambda g: (g, 0, 0)),
            pl.BlockSpec((1, RN, RN), lambda g: (g, 0, 0)),
            pl.BlockSpec((3, RN, LAT), lambda g: (0, 0, 0)),
            pl.BlockSpec((1, LAT), lambda g: (0, 0)),
            pl.BlockSpec((1, LAT), lambda g: (0, 0)),
            pl.BlockSpec((3, LAT, LAT), lambda g: (0, 0, 0)),
            pl.BlockSpec((1, LAT), lambda g: (0, 0)),
            pl.BlockSpec((1, LAT), lambda g: (0, 0)),
        ],
        out_specs=pl.BlockSpec((1, 1, 256), lambda g: (g, 0, 0)),
        out_shape=jax.ShapeDtypeStruct((NB, 1, 256), jnp.float32),
    )(A_all, X_all, w1, b1, pw1, w2, b2, pw2).reshape(NB, 256)


def _head_body(h_ref, fcw_ref, fcb_ref, bng_ref, bnb_ref, c1_ref, c2_ref,
               c2b_ref, f_ref, o_ref):
    h = jax.nn.relu(_dotl(h_ref[...], fcw_ref[...]) + fcb_ref[...])
    for v in range(2):
        hv = h[v * N_GR:(v + 1) * N_GR]
        mu = jnp.sum(hv, axis=0, keepdims=True) / float(N_GR)
        d = hv - mu
        var = jnp.sum(d * d, axis=0, keepdims=True) / float(N_GR)
        hn = d * lax.rsqrt(var + 1e-5) * bng_ref[...] + bnb_ref[...]
        out = _dotl(jax.nn.relu(_dotl(hn, c1_ref[...])), c2_ref[...]) + c2b_ref[...]
        fn = jnp.maximum(jnp.sqrt(jnp.sum(hn * hn, axis=1, keepdims=True)), 1e-12)
        on = jnp.maximum(jnp.sqrt(jnp.sum(out * out, axis=1, keepdims=True)), 1e-12)
        f_ref[v * N_GR:(v + 1) * N_GR, :] = hn / fn
        o_ref[v * N_GR:(v + 1) * N_GR, :] = out / on


def _head_tc(H, fcw, fcb, bng, bnb, c1w, c2w, c2b):
    return pl.pallas_call(
        _head_body,
        out_shape=(
            jax.ShapeDtypeStruct((NB, 256), jnp.float32),
            jax.ShapeDtypeStruct((NB, 512), jnp.float32),
        ),
    )(H, fcw, fcb, bng, bnb, c1w, c2w, c2b)


def _edge_blocks(edge_index, edge_attr):
    src = edge_index[0].reshape(N_GR, EPER)
    dst = edge_index[1].reshape(N_GR, EPER)
    g = jnp.arange(N_GR, dtype=jnp.int32)[:, None]
    flat = RN * src + dst - (RN * RN + RN) * g + g * REG
    flat = jnp.pad(flat, ((0, 0), (0, EPAD - EPER)),
                   constant_values=ASZ)
    vals = jnp.pad(edge_attr.reshape(N_GR, EPER), ((0, 0), (0, EPAD - EPER)))
    return (flat.astype(jnp.int32).reshape(N_GR, NCHUNK, CW),
            vals.reshape(N_GR, NCHUNK, CW))


def kernel(x1, edge_index1, edge_attr1, batch1,
           x2, edge_index2, edge_attr2, batch2, params):
    i1, v1 = _edge_blocks(edge_index1, edge_attr1)
    i2, v2 = _edge_blocks(edge_index2, edge_attr2)
    idx = jnp.concatenate([i1, i2], axis=0)
    vals = jnp.concatenate([v1, v2], axis=0)
    zeros = jnp.zeros((ASZ,), jnp.float32)

    A_all = _build_adj(idx, vals, zeros).reshape(NB, RN, RN)

    X_all = jnp.concatenate([x1.reshape(N_GR, RN, RN),
                             x2.reshape(N_GR, RN, RN)], axis=0)

    p = params
    H = _graph_tc(A_all, X_all,
                  p['conv1_W'], p['conv1_b'].reshape(1, LAT),
                  p['pool1_w'].reshape(1, LAT),
                  p['conv2_W'], p['conv2_b'].reshape(1, LAT),
                  p['pool2_w'].reshape(1, LAT))

    f_all, o_all = _head_tc(H, p['fc_W'], p['fc_b'].reshape(1, 256),
                            p['bn_g'].reshape(1, 256), p['bn_b'].reshape(1, 256),
                            p['c1_W'], p['c2_W'], p['c2_b'].reshape(1, 512))

    return (o_all[:N_GR], o_all[N_GR:], f_all[:N_GR], f_all[N_GR:])


# speedup vs baseline: 36.5798x; 1.2040x over previous
import functools

import jax
import jax.numpy as jnp
from jax import lax
from jax.experimental import pallas as pl
from jax.experimental.pallas import tpu as pltpu
from jax.experimental.pallas import tpu_sc as plsc

N_GR = 16
RN = 200
EPER = 8000
NB = 2 * N_GR
ASZ = RN * RN
REG = ASZ + 8
CW = 128
NCHUNK = (EPER + CW - 1) // CW
EPAD = NCHUNK * CW
K1 = RN // 2
K2 = K1 // 2
LAT = 64

_HI = lax.Precision.HIGHEST


def _dot(a, b):
    return jnp.dot(a, b, precision=_HI, preferred_element_type=jnp.float32)


def _dotl(a, b):
    return jnp.dot(a.astype(jnp.bfloat16), b.astype(jnp.bfloat16),
                   preferred_element_type=jnp.float32)


def _adj_body(idx_hbm, vals_hbm, zeros_hbm, out_hbm, idx_v, vals_v, buf_v,
              acc_sh, sem):
    c = lax.axis_index("c")
    s = lax.axis_index("s")
    b = c * N_GR + s
    base = s * REG

    pltpu.sync_copy(idx_hbm.at[b], idx_v)
    pltpu.sync_copy(vals_hbm.at[b], vals_v)
    pltpu.sync_copy(zeros_hbm, buf_v)
    pltpu.sync_copy(buf_v, acc_sh.at[pl.ds(base, ASZ)])

    def fire(j, carry):
        pltpu.async_copy(vals_v.at[j], acc_sh.at[idx_v.at[j]], sem,
                         add=True).wait()
        return carry

    lax.fori_loop(0, NCHUNK, fire, 0)

    pltpu.sync_copy(acc_sh.at[pl.ds(base, ASZ)], buf_v)
    pltpu.sync_copy(buf_v, out_hbm.at[b])


def _build_adj(idx, vals, zeros):
    mesh = plsc.VectorSubcoreMesh(core_axis_name="c", subcore_axis_name="s")
    f = pl.kernel(
        _adj_body,
        out_type=jax.ShapeDtypeStruct((NB, ASZ), jnp.float32),
        mesh=mesh,
        scratch_types=[
            pltpu.VMEM((NCHUNK, CW), jnp.int32),
            pltpu.VMEM((NCHUNK, CW), jnp.float32),
            pltpu.VMEM((ASZ,), jnp.float32),
            pltpu.VMEM_SHARED((N_GR * REG,), jnp.float32),
            pltpu.SemaphoreType.DMA,
        ],
    )
    return f(idx, vals, zeros)


def _col2row(v, n):
    del n
    return jnp.swapaxes(v, 0, 1)


def _cheb(A_norm, x, Wf, b_row, lmul_dot):
    tx1 = -lmul_dot(A_norm, x)
    tx2 = -2.0 * lmul_dot(A_norm, tx1) - x
    return _dotl(jnp.concatenate([x, tx1, tx2], axis=1), Wf) + b_row


def _norm_adj(A, n):
    deg = jnp.sum(A, axis=1, keepdims=True)
    pos = deg > 0.0
    dinv = jnp.where(pos, lax.rsqrt(jnp.where(pos, deg, 1.0)), 0.0)
    return A * dinv * _col2row(dinv, n)


def _select(score, n, k):
    i = lax.broadcasted_iota(jnp.int32, (n, n), 0)
    j = lax.broadcasted_iota(jnp.int32, (n, n), 1)
    s_row = _col2row(score, n)
    beats = (s_row > score) | ((s_row == score) & (j < i))
    rank = jnp.sum(beats.astype(jnp.float32), axis=1, keepdims=True)
    kept = rank < float(k)
    keptf = kept.astype(jnp.float32)
    lstrict = (j < i).astype(jnp.float32)
    posn = _dot(lstrict, keptf)
    pos_row = _col2row(posn, n)
    kept_row = _col2row(keptf, n)
    pk = lax.broadcasted_iota(jnp.int32, (k, 1), 0).astype(jnp.float32)
    S = ((pk == pos_row) & (kept_row > 0.5)).astype(jnp.float32)
    rk = lax.broadcasted_iota(jnp.int32, (1, k), 1).astype(jnp.float32)
    St = ((posn == rk) & kept).astype(jnp.float32)
    return S, St


def _sigmoid(z):
    return 1.0 / (1.0 + jnp.exp(-z))


def _score(x, pw_ref, n):
    pw = pw_ref[...]
    wnorm = jnp.sqrt(jnp.sum(pw * pw))
    pw_col = jnp.swapaxes(pw, 0, 1)
    return _sigmoid(_dotl(x, pw_col) / wnorm)


GPB = 4


def _graph_body(a_ref, x_ref, w1_ref, b1_ref, pw1_ref, w2_ref, b2_ref, pw2_ref,
                h_ref):
    ik = lax.broadcasted_iota(jnp.int32, (K1, K1), 0)
    jk = lax.broadcasted_iota(jnp.int32, (K1, K1), 1)
    eye_k = (ik == jk).astype(jnp.float32)
    for t in range(GPB):
        A = a_ref[t]
        x = x_ref[t]

        xc = _cheb(_norm_adj(A, RN), x, w1_ref[...], b1_ref[...], _dot)

        s1 = _score(xc, pw1_ref, RN)
        S1, St1 = _select(s1, RN, K1)
        xp = _dot(S1, xc * s1)
        Ap = _dot(S1, _dot(A, St1))

        h_ref[t, 0:1, 0:64] = jnp.max(xp, axis=0, keepdims=True)
        h_ref[t, 0:1, 64:128] = jnp.sum(xp, axis=0, keepdims=True) / float(K1)

        aaug = Ap + eye_k
        A2 = _dotl(aaug, aaug) * (1.0 - eye_k)

        xc2 = _cheb(_norm_adj(A2, K1), xp, w2_ref[...], b2_ref[...], _dotl)

        s2 = _score(xc2, pw2_ref, K1)
        S2, _ = _select(s2, K1, K2)
        xp2 = _dot(S2, xc2 * s2)

        h_ref[t, 0:1, 128:192] = jnp.max(xp2, axis=0, keepdims=True)
        h_ref[t, 0:1, 192:256] = jnp.sum(xp2, axis=0, keepdims=True) / float(K2)


def _graph_tc(A_all, X_all, w1, b1, pw1, w2, b2, pw2):
    return pl.pallas_call(
        _graph_body,
        grid=(NB // GPB,),
        in_specs=[
            pl.BlockSpec((GPB, RN, RN), lambda g: (g, 0, 0)),
            pl.BlockSpec((GPB, RN, RN), lambda g: (g, 0, 0)),
            pl.BlockSpec((3 * RN, LAT), lambda g: (0, 0)),
            pl.BlockSpec((1, LAT), lambda g: (0, 0)),
            pl.BlockSpec((1, LAT), lambda g: (0, 0)),
            pl.BlockSpec((3 * LAT, LAT), lambda g: (0, 0)),
            pl.BlockSpec((1, LAT), lambda g: (0, 0)),
            pl.BlockSpec((1, LAT), lambda g: (0, 0)),
        ],
        out_specs=pl.BlockSpec((GPB, 1, 256), lambda g: (g, 0, 0)),
        out_shape=jax.ShapeDtypeStruct((NB, 1, 256), jnp.float32),
    )(A_all, X_all, w1, b1, pw1, w2, b2, pw2).reshape(NB, 256)


def _head_body(h_ref, fcw_ref, fcb_ref, bng_ref, bnb_ref, c1_ref, c2_ref,
               c2b_ref, f_ref, o_ref):
    h = jax.nn.relu(_dotl(h_ref[...], fcw_ref[...]) + fcb_ref[...])
    for v in range(2):
        hv = h[v * N_GR:(v + 1) * N_GR]
        mu = jnp.sum(hv, axis=0, keepdims=True) / float(N_GR)
        d = hv - mu
        var = jnp.sum(d * d, axis=0, keepdims=True) / float(N_GR)
        hn = d * lax.rsqrt(var + 1e-5) * bng_ref[...] + bnb_ref[...]
        out = _dotl(jax.nn.relu(_dotl(hn, c1_ref[...])), c2_ref[...]) + c2b_ref[...]
        fn = jnp.maximum(jnp.sqrt(jnp.sum(hn * hn, axis=1, keepdims=True)), 1e-12)
        on = jnp.maximum(jnp.sqrt(jnp.sum(out * out, axis=1, keepdims=True)), 1e-12)
        f_ref[v * N_GR:(v + 1) * N_GR, :] = hn / fn
        o_ref[v * N_GR:(v + 1) * N_GR, :] = out / on


def _head_tc(H, fcw, fcb, bng, bnb, c1w, c2w, c2b):
    return pl.pallas_call(
        _head_body,
        out_shape=(
            jax.ShapeDtypeStruct((NB, 256), jnp.float32),
            jax.ShapeDtypeStruct((NB, 512), jnp.float32),
        ),
    )(H, fcw, fcb, bng, bnb, c1w, c2w, c2b)


def _edge_blocks(edge_index, edge_attr):
    src = edge_index[0].reshape(N_GR, EPER)
    dst = edge_index[1].reshape(N_GR, EPER)
    g = jnp.arange(N_GR, dtype=jnp.int32)[:, None]
    flat = RN * src + dst - (RN * RN + RN) * g + g * REG
    flat = jnp.pad(flat, ((0, 0), (0, EPAD - EPER)),
                   constant_values=ASZ)
    vals = jnp.pad(edge_attr.reshape(N_GR, EPER), ((0, 0), (0, EPAD - EPER)))
    return (flat.astype(jnp.int32).reshape(N_GR, NCHUNK, CW),
            vals.reshape(N_GR, NCHUNK, CW))


def kernel(x1, edge_index1, edge_attr1, batch1,
           x2, edge_index2, edge_attr2, batch2, params):
    i1, v1 = _edge_blocks(edge_index1, edge_attr1)
    i2, v2 = _edge_blocks(edge_index2, edge_attr2)
    idx = jnp.concatenate([i1, i2], axis=0)
    vals = jnp.concatenate([v1, v2], axis=0)
    zeros = jnp.zeros((ASZ,), jnp.float32)

    A_all = _build_adj(idx, vals, zeros).reshape(NB, RN, RN)

    X_all = jnp.concatenate([x1.reshape(N_GR, RN, RN),
                             x2.reshape(N_GR, RN, RN)], axis=0)

    p = params
    H = _graph_tc(A_all, X_all,
                  p['conv1_W'].reshape(3 * RN, LAT),
                  p['conv1_b'].reshape(1, LAT),
                  p['pool1_w'].reshape(1, LAT),
                  p['conv2_W'].reshape(3 * LAT, LAT),
                  p['conv2_b'].reshape(1, LAT),
                  p['pool2_w'].reshape(1, LAT))

    f_all, o_all = _head_tc(H, p['fc_W'], p['fc_b'].reshape(1, 256),
                            p['bn_g'].reshape(1, 256), p['bn_b'].reshape(1, 256),
                            p['c1_W'], p['c2_W'], p['c2_b'].reshape(1, 512))

    return (o_all[:N_GR], o_all[N_GR:], f_all[:N_GR], f_all[N_GR:])


# speedup vs baseline: 36.6422x; 1.0017x over previous
import functools

import jax
import jax.numpy as jnp
from jax import lax
from jax.experimental import pallas as pl
from jax.experimental.pallas import tpu as pltpu
from jax.experimental.pallas import tpu_sc as plsc

N_GR = 16
RN = 200
EPER = 8000
NB = 2 * N_GR
ASZ = RN * RN
REG = ASZ + 8
CW = 128
NCHUNK = (EPER + CW - 1) // CW
EPAD = NCHUNK * CW
K1 = RN // 2
K2 = K1 // 2
LAT = 64

_HI = lax.Precision.HIGHEST


def _dot(a, b):
    return jnp.dot(a, b, precision=_HI, preferred_element_type=jnp.float32)


def _dotl(a, b):
    return jnp.dot(a.astype(jnp.bfloat16), b.astype(jnp.bfloat16),
                   preferred_element_type=jnp.float32)


def _adj_body(idx_hbm, vals_hbm, zeros_hbm, out_hbm, idx_v, vals_v, buf_v,
              acc_sh, sem):
    c = lax.axis_index("c")
    s = lax.axis_index("s")
    b = c * N_GR + s
    base = s * REG

    pltpu.sync_copy(idx_hbm.at[b], idx_v)
    pltpu.sync_copy(vals_hbm.at[b], vals_v)
    pltpu.sync_copy(zeros_hbm, buf_v)
    pltpu.sync_copy(buf_v, acc_sh.at[pl.ds(base, ASZ)])

    pltpu.async_copy(vals_v, acc_sh.at[idx_v], sem, add=True).wait()

    pltpu.sync_copy(acc_sh.at[pl.ds(base, ASZ)], buf_v)
    pltpu.sync_copy(buf_v, out_hbm.at[b])


def _build_adj(idx, vals, zeros):
    mesh = plsc.VectorSubcoreMesh(core_axis_name="c", subcore_axis_name="s")
    f = pl.kernel(
        _adj_body,
        out_type=jax.ShapeDtypeStruct((NB, ASZ), jnp.float32),
        mesh=mesh,
        scratch_types=[
            pltpu.VMEM((EPAD,), jnp.int32),
            pltpu.VMEM((EPAD,), jnp.float32),
            pltpu.VMEM((ASZ,), jnp.float32),
            pltpu.VMEM_SHARED((N_GR * REG,), jnp.float32),
            pltpu.SemaphoreType.DMA,
        ],
    )
    return f(idx, vals, zeros)


def _col2row(v, n):
    del n
    return jnp.swapaxes(v, 0, 1)


def _cheb(A_norm, x, Wf, b_row, lmul_dot):
    tx1 = -lmul_dot(A_norm, x)
    tx2 = -2.0 * lmul_dot(A_norm, tx1) - x
    return _dotl(jnp.concatenate([x, tx1, tx2], axis=1), Wf) + b_row


def _norm_adj(A, n):
    deg = jnp.sum(A, axis=1, keepdims=True)
    pos = deg > 0.0
    dinv = jnp.where(pos, lax.rsqrt(jnp.where(pos, deg, 1.0)), 0.0)
    return A * dinv * _col2row(dinv, n)


def _select(score, n, k):
    i = lax.broadcasted_iota(jnp.int32, (n, n), 0)
    j = lax.broadcasted_iota(jnp.int32, (n, n), 1)
    s_row = _col2row(score, n)
    beats = (s_row > score) | ((s_row == score) & (j < i))
    rank = jnp.sum(beats.astype(jnp.float32), axis=1, keepdims=True)
    kept = rank < float(k)
    keptf = kept.astype(jnp.float32)
    lstrict = (j < i).astype(jnp.float32)
    posn = _dot(lstrict, keptf)
    pos_row = _col2row(posn, n)
    kept_row = _col2row(keptf, n)
    pk = lax.broadcasted_iota(jnp.int32, (k, 1), 0).astype(jnp.float32)
    S = ((pk == pos_row) & (kept_row > 0.5)).astype(jnp.float32)
    rk = lax.broadcasted_iota(jnp.int32, (1, k), 1).astype(jnp.float32)
    St = ((posn == rk) & kept).astype(jnp.float32)
    return S, St


def _sigmoid(z):
    return 1.0 / (1.0 + jnp.exp(-z))


def _score(x, pw_ref, n):
    pw = pw_ref[...]
    wnorm = jnp.sqrt(jnp.sum(pw * pw))
    pw_col = jnp.swapaxes(pw, 0, 1)
    return _sigmoid(_dotl(x, pw_col) / wnorm)


GPB = 4


def _graph_body(a_ref, x_ref, w1_ref, b1_ref, pw1_ref, w2_ref, b2_ref, pw2_ref,
                h_ref):
    ik = lax.broadcasted_iota(jnp.int32, (K1, K1), 0)
    jk = lax.broadcasted_iota(jnp.int32, (K1, K1), 1)
    eye_k = (ik == jk).astype(jnp.float32)
    for t in range(GPB):
        A = a_ref[t]
        x = x_ref[t]

        xc = _cheb(_norm_adj(A, RN), x, w1_ref[...], b1_ref[...], _dot)

        s1 = _score(xc, pw1_ref, RN)
        S1, St1 = _select(s1, RN, K1)
        xp = _dot(S1, xc * s1)
        Ap = _dot(S1, _dot(A, St1))

        h_ref[t, 0:1, 0:64] = jnp.max(xp, axis=0, keepdims=True)
        h_ref[t, 0:1, 64:128] = jnp.sum(xp, axis=0, keepdims=True) / float(K1)

        aaug = Ap + eye_k
        A2 = _dotl(aaug, aaug) * (1.0 - eye_k)

        xc2 = _cheb(_norm_adj(A2, K1), xp, w2_ref[...], b2_ref[...], _dotl)

        s2 = _score(xc2, pw2_ref, K1)
        S2, _ = _select(s2, K1, K2)
        xp2 = _dot(S2, xc2 * s2)

        h_ref[t, 0:1, 128:192] = jnp.max(xp2, axis=0, keepdims=True)
        h_ref[t, 0:1, 192:256] = jnp.sum(xp2, axis=0, keepdims=True) / float(K2)


def _graph_tc(A_all, X_all, w1, b1, pw1, w2, b2, pw2):
    return pl.pallas_call(
        _graph_body,
        grid=(NB // GPB,),
        in_specs=[
            pl.BlockSpec((GPB, RN, RN), lambda g: (g, 0, 0)),
            pl.BlockSpec((GPB, RN, RN), lambda g: (g, 0, 0)),
            pl.BlockSpec((3 * RN, LAT), lambda g: (0, 0)),
            pl.BlockSpec((1, LAT), lambda g: (0, 0)),
            pl.BlockSpec((1, LAT), lambda g: (0, 0)),
            pl.BlockSpec((3 * LAT, LAT), lambda g: (0, 0)),
            pl.BlockSpec((1, LAT), lambda g: (0, 0)),
            pl.BlockSpec((1, LAT), lambda g: (0, 0)),
        ],
        out_specs=pl.BlockSpec((GPB, 1, 256), lambda g: (g, 0, 0)),
        out_shape=jax.ShapeDtypeStruct((NB, 1, 256), jnp.float32),
    )(A_all, X_all, w1, b1, pw1, w2, b2, pw2).reshape(NB, 256)


def _head_body(h_ref, fcw_ref, fcb_ref, bng_ref, bnb_ref, c1_ref, c2_ref,
               c2b_ref, f_ref, o_ref):
    h = jax.nn.relu(_dotl(h_ref[...], fcw_ref[...]) + fcb_ref[...])
    for v in range(2):
        hv = h[v * N_GR:(v + 1) * N_GR]
        mu = jnp.sum(hv, axis=0, keepdims=True) / float(N_GR)
        d = hv - mu
        var = jnp.sum(d * d, axis=0, keepdims=True) / float(N_GR)
        hn = d * lax.rsqrt(var + 1e-5) * bng_ref[...] + bnb_ref[...]
        out = _dotl(jax.nn.relu(_dotl(hn, c1_ref[...])), c2_ref[...]) + c2b_ref[...]
        fn = jnp.maximum(jnp.sqrt(jnp.sum(hn * hn, axis=1, keepdims=True)), 1e-12)
        on = jnp.maximum(jnp.sqrt(jnp.sum(out * out, axis=1, keepdims=True)), 1e-12)
        f_ref[v * N_GR:(v + 1) * N_GR, :] = hn / fn
        o_ref[v * N_GR:(v + 1) * N_GR, :] = out / on


def _head_tc(H, fcw, fcb, bng, bnb, c1w, c2w, c2b):
    return pl.pallas_call(
        _head_body,
        out_shape=(
            jax.ShapeDtypeStruct((NB, 256), jnp.float32),
            jax.ShapeDtypeStruct((NB, 512), jnp.float32),
        ),
    )(H, fcw, fcb, bng, bnb, c1w, c2w, c2b)


def _edge_blocks(edge_index, edge_attr):
    src = edge_index[0].reshape(N_GR, EPER)
    dst = edge_index[1].reshape(N_GR, EPER)
    g = jnp.arange(N_GR, dtype=jnp.int32)[:, None]
    flat = RN * src + dst - (RN * RN + RN) * g + g * REG
    flat = jnp.pad(flat, ((0, 0), (0, EPAD - EPER)),
                   constant_values=ASZ)
    vals = jnp.pad(edge_attr.reshape(N_GR, EPER), ((0, 0), (0, EPAD - EPER)))
    return flat.astype(jnp.int32), vals


def kernel(x1, edge_index1, edge_attr1, batch1,
           x2, edge_index2, edge_attr2, batch2, params):
    i1, v1 = _edge_blocks(edge_index1, edge_attr1)
    i2, v2 = _edge_blocks(edge_index2, edge_attr2)
    idx = jnp.concatenate([i1, i2], axis=0)
    vals = jnp.concatenate([v1, v2], axis=0)
    zeros = jnp.zeros((ASZ,), jnp.float32)

    A_all = _build_adj(idx, vals, zeros).reshape(NB, RN, RN)

    X_all = jnp.concatenate([x1.reshape(N_GR, RN, RN),
                             x2.reshape(N_GR, RN, RN)], axis=0)

    p = params
    H = _graph_tc(A_all, X_all,
                  p['conv1_W'].reshape(3 * RN, LAT),
                  p['conv1_b'].reshape(1, LAT),
                  p['pool1_w'].reshape(1, LAT),
                  p['conv2_W'].reshape(3 * LAT, LAT),
                  p['conv2_b'].reshape(1, LAT),
                  p['pool2_w'].reshape(1, LAT))

    f_all, o_all = _head_tc(H, p['fc_W'], p['fc_b'].reshape(1, 256),
                            p['bn_g'].reshape(1, 256), p['bn_b'].reshape(1, 256),
                            p['c1_W'], p['c2_W'], p['c2_b'].reshape(1, 512))

    return (o_all[:N_GR], o_all[N_GR:], f_all[:N_GR], f_all[N_GR:])


# speedup vs baseline: 51.8103x; 1.4140x over previous
import functools

import jax
import jax.numpy as jnp
from jax import lax
from jax.experimental import pallas as pl
from jax.experimental.pallas import tpu as pltpu
from jax.experimental.pallas import tpu_sc as plsc

N_GR = 16
RN = 200
EPER = 8000
NB = 2 * N_GR
ASZ = RN * RN
REG = ASZ + 8
CW = 128
NCHUNK = (EPER + CW - 1) // CW
EPAD = NCHUNK * CW
K1 = RN // 2
K2 = K1 // 2
LAT = 64

_HI = lax.Precision.HIGHEST


def _dot(a, b):
    return jnp.dot(a, b, precision=_HI, preferred_element_type=jnp.float32)


def _dotl(a, b):
    return jnp.dot(a.astype(jnp.bfloat16), b.astype(jnp.bfloat16),
                   preferred_element_type=jnp.float32)


def _adj_body(idx_hbm, vals_hbm, zeros_hbm, out_hbm, idx_v, vals_v, buf_v,
              acc_sh, sem):
    c = lax.axis_index("c")
    s = lax.axis_index("s")
    b = c * N_GR + s
    base = s * REG

    pltpu.sync_copy(idx_hbm.at[b], idx_v)
    pltpu.sync_copy(vals_hbm.at[b], vals_v)
    pltpu.sync_copy(zeros_hbm, buf_v)
    pltpu.sync_copy(buf_v, acc_sh.at[pl.ds(base, ASZ)])

    pltpu.async_copy(vals_v, acc_sh.at[idx_v], sem, add=True).wait()

    pltpu.sync_copy(acc_sh.at[pl.ds(base, ASZ)], buf_v)
    pltpu.sync_copy(buf_v, out_hbm.at[b])


def _build_adj(idx, vals, zeros):
    mesh = plsc.VectorSubcoreMesh(core_axis_name="c", subcore_axis_name="s")
    f = pl.kernel(
        _adj_body,
        out_type=jax.ShapeDtypeStruct((NB, ASZ), jnp.float32),
        mesh=mesh,
        scratch_types=[
            pltpu.VMEM((EPAD,), jnp.int32),
            pltpu.VMEM((EPAD,), jnp.float32),
            pltpu.VMEM((ASZ,), jnp.float32),
            pltpu.VMEM_SHARED((N_GR * REG,), jnp.float32),
            pltpu.SemaphoreType.DMA,
        ],
    )
    return f(idx, vals, zeros)


def _col2row(v, n):
    del n
    return jnp.swapaxes(v, 0, 1)


def _cheb(A_norm, x, Wf, b_row, lmul_dot):
    tx1 = -lmul_dot(A_norm, x)
    tx2 = -2.0 * lmul_dot(A_norm, tx1) - x
    return _dotl(jnp.concatenate([x, tx1, tx2], axis=1), Wf) + b_row


def _norm_adj(A, n):
    deg = jnp.sum(A, axis=1, keepdims=True)
    pos = deg > 0.0
    dinv = jnp.where(pos, lax.rsqrt(jnp.where(pos, deg, 1.0)), 0.0)
    return A * dinv * _col2row(dinv, n)


def _select_mask(score, n, k):
    i = lax.broadcasted_iota(jnp.int32, (n, n), 0)
    j = lax.broadcasted_iota(jnp.int32, (n, n), 1)
    s_row = _col2row(score, n)
    beats = (s_row > score) | ((s_row == score) & (j < i))
    rank = jnp.sum(beats.astype(jnp.float32), axis=1, keepdims=True)
    return (rank < float(k)).astype(jnp.float32)


def _sigmoid(z):
    return 1.0 / (1.0 + jnp.exp(-z))


def _score(x, pw_ref, n):
    pw = pw_ref[...]
    wnorm = jnp.sqrt(jnp.sum(pw * pw))
    pw_col = jnp.swapaxes(pw, 0, 1)
    return _sigmoid(_dotl(x, pw_col) / wnorm)


GPB = 4

_NEG = -3.0e38


def _graph_body(a_ref, x_ref, w1_ref, b1_ref, pw1_ref, w2_ref, b2_ref, pw2_ref,
                h_ref):
    inn = lax.broadcasted_iota(jnp.int32, (RN, RN), 0)
    jnn = lax.broadcasted_iota(jnp.int32, (RN, RN), 1)
    eye_n = (inn == jnn).astype(jnp.float32)
    for t in range(GPB):
        A = a_ref[t]
        x = x_ref[t]

        xc = _cheb(_norm_adj(A, RN), x, w1_ref[...], b1_ref[...], _dot)

        s1 = _score(xc, pw1_ref, RN)
        k1 = _select_mask(s1, RN, K1)
        xp = xc * s1 * k1

        h_ref[t, 0:1, 0:64] = jnp.max(
            jnp.where(k1 > 0.5, xp, _NEG), axis=0, keepdims=True)
        h_ref[t, 0:1, 64:128] = jnp.sum(xp, axis=0, keepdims=True) / float(K1)

        aaug = A * k1 * _col2row(k1, RN) + eye_n * k1
        A2 = _dotl(aaug, aaug) * (1.0 - eye_n)

        xc2 = _cheb(_norm_adj(A2, RN), xp, w2_ref[...], b2_ref[...], _dotl)

        s2 = _score(xc2, pw2_ref, RN)
        k2 = _select_mask(jnp.where(k1 > 0.5, s2, -1.0), RN, K2)
        xp2 = xc2 * s2 * k2

        h_ref[t, 0:1, 128:192] = jnp.max(
            jnp.where(k2 > 0.5, xp2, _NEG), axis=0, keepdims=True)
        h_ref[t, 0:1, 192:256] = jnp.sum(xp2, axis=0, keepdims=True) / float(K2)


def _graph_tc(A_all, X_all, w1, b1, pw1, w2, b2, pw2):
    return pl.pallas_call(
        _graph_body,
        grid=(NB // GPB,),
        in_specs=[
            pl.BlockSpec((GPB, RN, RN), lambda g: (g, 0, 0)),
            pl.BlockSpec((GPB, RN, RN), lambda g: (g, 0, 0)),
            pl.BlockSpec((3 * RN, LAT), lambda g: (0, 0)),
            pl.BlockSpec((1, LAT), lambda g: (0, 0)),
            pl.BlockSpec((1, LAT), lambda g: (0, 0)),
            pl.BlockSpec((3 * LAT, LAT), lambda g: (0, 0)),
            pl.BlockSpec((1, LAT), lambda g: (0, 0)),
            pl.BlockSpec((1, LAT), lambda g: (0, 0)),
        ],
        out_specs=pl.BlockSpec((GPB, 1, 256), lambda g: (g, 0, 0)),
        out_shape=jax.ShapeDtypeStruct((NB, 1, 256), jnp.float32),
    )(A_all, X_all, w1, b1, pw1, w2, b2, pw2).reshape(NB, 256)


def _head_body(h_ref, fcw_ref, fcb_ref, bng_ref, bnb_ref, c1_ref, c2_ref,
               c2b_ref, f_ref, o_ref):
    h = jax.nn.relu(_dotl(h_ref[...], fcw_ref[...]) + fcb_ref[...])
    for v in range(2):
        hv = h[v * N_GR:(v + 1) * N_GR]
        mu = jnp.sum(hv, axis=0, keepdims=True) / float(N_GR)
        d = hv - mu
        var = jnp.sum(d * d, axis=0, keepdims=True) / float(N_GR)
        hn = d * lax.rsqrt(var + 1e-5) * bng_ref[...] + bnb_ref[...]
        out = _dotl(jax.nn.relu(_dotl(hn, c1_ref[...])), c2_ref[...]) + c2b_ref[...]
        fn = jnp.maximum(jnp.sqrt(jnp.sum(hn * hn, axis=1, keepdims=True)), 1e-12)
        on = jnp.maximum(jnp.sqrt(jnp.sum(out * out, axis=1, keepdims=True)), 1e-12)
        f_ref[v * N_GR:(v + 1) * N_GR, :] = hn / fn
        o_ref[v * N_GR:(v + 1) * N_GR, :] = out / on


def _head_tc(H, fcw, fcb, bng, bnb, c1w, c2w, c2b):
    return pl.pallas_call(
        _head_body,
        out_shape=(
            jax.ShapeDtypeStruct((NB, 256), jnp.float32),
            jax.ShapeDtypeStruct((NB, 512), jnp.float32),
        ),
    )(H, fcw, fcb, bng, bnb, c1w, c2w, c2b)


def _edge_blocks(edge_index, edge_attr):
    src = edge_index[0].reshape(N_GR, EPER)
    dst = edge_index[1].reshape(N_GR, EPER)
    g = jnp.arange(N_GR, dtype=jnp.int32)[:, None]
    flat = RN * src + dst - (RN * RN + RN) * g + g * REG
    flat = jnp.pad(flat, ((0, 0), (0, EPAD - EPER)),
                   constant_values=ASZ)
    vals = jnp.pad(edge_attr.reshape(N_GR, EPER), ((0, 0), (0, EPAD - EPER)))
    return flat.astype(jnp.int32), vals


def kernel(x1, edge_index1, edge_attr1, batch1,
           x2, edge_index2, edge_attr2, batch2, params):
    i1, v1 = _edge_blocks(edge_index1, edge_attr1)
    i2, v2 = _edge_blocks(edge_index2, edge_attr2)
    idx = jnp.concatenate([i1, i2], axis=0)
    vals = jnp.concatenate([v1, v2], axis=0)
    zeros = jnp.zeros((ASZ,), jnp.float32)

    A_all = _build_adj(idx, vals, zeros).reshape(NB, RN, RN)

    X_all = jnp.concatenate([x1.reshape(N_GR, RN, RN),
                             x2.reshape(N_GR, RN, RN)], axis=0)

    p = params
    H = _graph_tc(A_all, X_all,
                  p['conv1_W'].reshape(3 * RN, LAT),
                  p['conv1_b'].reshape(1, LAT),
                  p['pool1_w'].reshape(1, LAT),
                  p['conv2_W'].reshape(3 * LAT, LAT),
                  p['conv2_b'].reshape(1, LAT),
                  p['pool2_w'].reshape(1, LAT))

    f_all, o_all = _head_tc(H, p['fc_W'], p['fc_b'].reshape(1, 256),
                            p['bn_g'].reshape(1, 256), p['bn_b'].reshape(1, 256),
                            p['c1_W'], p['c2_W'], p['c2_b'].reshape(1, 512))

    return (o_all[:N_GR], o_all[N_GR:], f_all[:N_GR], f_all[N_GR:])


# speedup vs baseline: 52.9877x; 1.0227x over previous
import functools

import jax
import jax.numpy as jnp
from jax import lax
from jax.experimental import pallas as pl
from jax.experimental.pallas import tpu as pltpu
from jax.experimental.pallas import tpu_sc as plsc

N_GR = 16
RN = 200
EPER = 8000
NB = 2 * N_GR
ASZ = RN * RN
REG = ASZ + 8
K1 = RN // 2
K2 = K1 // 2
LAT = 64

_HI = lax.Precision.HIGHEST


def _dot(a, b):
    return jnp.dot(a, b, precision=_HI, preferred_element_type=jnp.float32)


def _dotl(a, b):
    return jnp.dot(a.astype(jnp.bfloat16), b.astype(jnp.bfloat16),
                   preferred_element_type=jnp.float32)


def _adj_body(idx1_hbm, vals1_hbm, idx2_hbm, vals2_hbm, zeros_hbm, out_hbm,
              idx_v, vals_v, buf_v, acc_sh, sem):
    c = lax.axis_index("c")
    s = lax.axis_index("s")
    b = c * N_GR + s
    base = s * REG

    @pl.when(c == 0)
    def _():
        pltpu.sync_copy(idx1_hbm.at[s], idx_v)
        pltpu.sync_copy(vals1_hbm.at[s], vals_v)

    @pl.when(c != 0)
    def _():
        pltpu.sync_copy(idx2_hbm.at[s], idx_v)
        pltpu.sync_copy(vals2_hbm.at[s], vals_v)

    pltpu.sync_copy(zeros_hbm, buf_v)
    pltpu.sync_copy(buf_v, acc_sh.at[pl.ds(base, ASZ)])

    pltpu.async_copy(vals_v, acc_sh.at[idx_v], sem, add=True).wait()

    pltpu.sync_copy(acc_sh.at[pl.ds(base, ASZ)], buf_v)
    pltpu.sync_copy(buf_v, out_hbm.at[b])


def _build_adj(idx1, vals1, idx2, vals2, zeros):
    mesh = plsc.VectorSubcoreMesh(core_axis_name="c", subcore_axis_name="s")
    f = pl.kernel(
        _adj_body,
        out_type=jax.ShapeDtypeStruct((NB, ASZ), jnp.float32),
        mesh=mesh,
        scratch_types=[
            pltpu.VMEM((EPER,), jnp.int32),
            pltpu.VMEM((EPER,), jnp.float32),
            pltpu.VMEM((ASZ,), jnp.float32),
            pltpu.VMEM_SHARED((N_GR * REG,), jnp.float32),
            pltpu.SemaphoreType.DMA,
        ],
    )
    return f(idx1, vals1, idx2, vals2, zeros)


def _col2row(v):
    return jnp.swapaxes(v, 0, 1)


def _cheb(A_norm, x, Wf, b_row, lmul_dot):
    tx1 = -lmul_dot(A_norm, x)
    tx2 = -2.0 * lmul_dot(A_norm, tx1) - x
    return _dotl(jnp.concatenate([x, tx1, tx2], axis=1), Wf) + b_row


def _norm_adj(A, n):
    deg = jnp.sum(A, axis=1, keepdims=True)
    pos = deg > 0.0
    dinv = jnp.where(pos, lax.rsqrt(jnp.where(pos, deg, 1.0)), 0.0)
    return A * dinv * _col2row(dinv)


def _select_mask(score, n, k):
    i = lax.broadcasted_iota(jnp.int32, (n, n), 0)
    j = lax.broadcasted_iota(jnp.int32, (n, n), 1)
    s_row = _col2row(score)
    beats = (s_row > score) | ((s_row == score) & (j < i))
    rank = jnp.sum(beats.astype(jnp.float32), axis=1, keepdims=True)
    return (rank < float(k)).astype(jnp.float32)


def _sigmoid(z):
    return 1.0 / (1.0 + jnp.exp(-z))


def _score(x, pw_ref):
    pw = pw_ref[...]
    wnorm = jnp.sqrt(jnp.sum(pw * pw))
    pw_col = jnp.swapaxes(pw, 0, 1)
    return _sigmoid(_dotl(x, pw_col) / wnorm)


GPB = 4

_NEG = -3.0e38


def _graph_body(a_ref, x_ref, w1_ref, b1_ref, pw1_ref, w2_ref, b2_ref, pw2_ref,
                h_ref):
    inn = lax.broadcasted_iota(jnp.int32, (RN, RN), 0)
    jnn = lax.broadcasted_iota(jnp.int32, (RN, RN), 1)
    eye_n = (inn == jnn).astype(jnp.float32)
    for t in range(GPB):
        A = a_ref[t]
        x = x_ref[t]

        xc = _cheb(_norm_adj(A, RN), x, w1_ref[...], b1_ref[...], _dot)

        s1 = _score(xc, pw1_ref)
        k1 = _select_mask(s1, RN, K1)
        xp = xc * s1 * k1

        h_ref[t, 0:1, 0:64] = jnp.max(
            jnp.where(k1 > 0.5, xp, _NEG), axis=0, keepdims=True)
        h_ref[t, 0:1, 64:128] = jnp.sum(xp, axis=0, keepdims=True) / float(K1)

        aaug = A * k1 * _col2row(k1) + eye_n * k1
        A2 = _dotl(aaug, aaug) * (1.0 - eye_n)

        xc2 = _cheb(_norm_adj(A2, RN), xp, w2_ref[...], b2_ref[...], _dotl)

        s2 = _score(xc2, pw2_ref)
        k2 = _select_mask(jnp.where(k1 > 0.5, s2, -1.0), RN, K2)
        xp2 = xc2 * s2 * k2

        h_ref[t, 0:1, 128:192] = jnp.max(
            jnp.where(k2 > 0.5, xp2, _NEG), axis=0, keepdims=True)
        h_ref[t, 0:1, 192:256] = jnp.sum(xp2, axis=0, keepdims=True) / float(K2)


def _graph_tc(A_all, X_all, w1, b1, pw1, w2, b2, pw2):
    return pl.pallas_call(
        _graph_body,
        grid=(NB // GPB,),
        in_specs=[
            pl.BlockSpec((GPB, RN, RN), lambda g: (g, 0, 0)),
            pl.BlockSpec((GPB, RN, RN), lambda g: (g, 0, 0)),
            pl.BlockSpec((3 * RN, LAT), lambda g: (0, 0)),
            pl.BlockSpec((1, LAT), lambda g: (0, 0)),
            pl.BlockSpec((1, LAT), lambda g: (0, 0)),
            pl.BlockSpec((3 * LAT, LAT), lambda g: (0, 0)),
            pl.BlockSpec((1, LAT), lambda g: (0, 0)),
            pl.BlockSpec((1, LAT), lambda g: (0, 0)),
        ],
        out_specs=pl.BlockSpec((GPB, 1, 256), lambda g: (g, 0, 0)),
        out_shape=jax.ShapeDtypeStruct((NB, 1, 256), jnp.float32),
    )(A_all, X_all, w1, b1, pw1, w2, b2, pw2).reshape(NB, 256)


def _head_body(h_ref, fcw_ref, fcb_ref, bng_ref, bnb_ref, c1_ref, c2_ref,
               c2b_ref, o1_ref, o2_ref, f1_ref, f2_ref):
    h = jax.nn.relu(_dotl(h_ref[...], fcw_ref[...]) + fcb_ref[...])
    for v, (f_ref, o_ref) in enumerate(((f1_ref, o1_ref), (f2_ref, o2_ref))):
        hv = h[v * N_GR:(v + 1) * N_GR]
        mu = jnp.sum(hv, axis=0, keepdims=True) / float(N_GR)
        d = hv - mu
        var = jnp.sum(d * d, axis=0, keepdims=True) / float(N_GR)
        hn = d * lax.rsqrt(var + 1e-5) * bng_ref[...] + bnb_ref[...]
        out = _dotl(jax.nn.relu(_dotl(hn, c1_ref[...])), c2_ref[...]) + c2b_ref[...]
        fn = jnp.maximum(jnp.sqrt(jnp.sum(hn * hn, axis=1, keepdims=True)), 1e-12)
        on = jnp.maximum(jnp.sqrt(jnp.sum(out * out, axis=1, keepdims=True)), 1e-12)
        f_ref[...] = hn / fn
        o_ref[...] = out / on


def _head_tc(H, fcw, fcb, bng, bnb, c1w, c2w, c2b):
    return pl.pallas_call(
        _head_body,
        out_shape=(
            jax.ShapeDtypeStruct((N_GR, 512), jnp.float32),
            jax.ShapeDtypeStruct((N_GR, 512), jnp.float32),
            jax.ShapeDtypeStruct((N_GR, 256), jnp.float32),
            jax.ShapeDtypeStruct((N_GR, 256), jnp.float32),
        ),
    )(H, fcw, fcb, bng, bnb, c1w, c2w, c2b)


def _edge_blocks(edge_index, edge_attr):
    src = edge_index[0].reshape(N_GR, EPER)
    dst = edge_index[1].reshape(N_GR, EPER)
    g = jnp.arange(N_GR, dtype=jnp.int32)[:, None]
    flat = RN * src + dst - (RN * RN + RN) * g + g * REG
    return flat.astype(jnp.int32), edge_attr.reshape(N_GR, EPER)


def kernel(x1, edge_index1, edge_attr1, batch1,
           x2, edge_index2, edge_attr2, batch2, params):
    i1, v1 = _edge_blocks(edge_index1, edge_attr1)
    i2, v2 = _edge_blocks(edge_index2, edge_attr2)
    zeros = jnp.zeros((ASZ,), jnp.float32)

    A_all = _build_adj(i1, v1, i2, v2, zeros).reshape(NB, RN, RN)

    X_all = jnp.concatenate([x1.reshape(N_GR, RN, RN),
                             x2.reshape(N_GR, RN, RN)], axis=0)

    p = params
    H = _graph_tc(A_all, X_all,
                  p['conv1_W'].reshape(3 * RN, LAT),
                  p['conv1_b'].reshape(1, LAT),
                  p['pool1_w'].reshape(1, LAT),
                  p['conv2_W'].reshape(3 * LAT, LAT),
                  p['conv2_b'].reshape(1, LAT),
                  p['pool2_w'].reshape(1, LAT))

    o1, o2, f1, f2 = _head_tc(H, p['fc_W'], p['fc_b'].reshape(1, 256),
                              p['bn_g'].reshape(1, 256),
                              p['bn_b'].reshape(1, 256),
                              p['c1_W'], p['c2_W'], p['c2_b'].reshape(1, 512))
    return (o1, o2, f1, f2)
